# baseline (speedup 1.0000x reference)
import functools

import jax
import jax.numpy as jnp
from jax import lax
from jax.experimental import pallas as pl
from jax.experimental.pallas import tpu as pltpu
from jax.experimental.pallas import tpu_sc as plsc

_W = 10000
_D = 2048
_B = 8192

_NC = 2
_NS = 16
_NW = _NC * _NS

_R0 = 9472
_CH = 16
_K = (_W - _R0 - _CH) // (_CH * _NW)
_LANES = 16
_JGRID = _D // _LANES


def _sc_body(buf_hbm, out_hbm, cbuf0, cbuf1, acc_v, spart, comb_v, out_v,
             sem0, sem1):
    c_ax = lax.axis_index("c")
    s_ax = lax.axis_index("s")
    w = s_ax * _NC + c_ax

    def zero_acc(j, _):
        acc_v[pl.ds(j * _LANES, _LANES)] = jnp.zeros((_LANES,), jnp.float32)
        return 0

    lax.fori_loop(0, _JGRID, zero_acc, 0)

    def start(k, dbuf, sem):
        row0 = _R0 + (w + _NW * k) * _CH
        return pltpu.async_copy(buf_hbm.at[pl.ds(row0, _CH), :], dbuf, sem)

    def accumulate(dbuf):
        def jbody(j, _):
            o = j * _LANES
            v = acc_v[pl.ds(o, _LANES)]
            for r in range(_CH):
                v = v + dbuf[r, pl.ds(o, _LANES)]
            acc_v[pl.ds(o, _LANES)] = v
            return 0

        lax.fori_loop(0, _JGRID, jbody, 0)

    bufs = (cbuf0, cbuf1)
    sems = (sem0, sem1)
    handles = [None] * _K
    handles[0] = start(0, bufs[0], sems[0])
    for k in range(_K):
        if k + 1 < _K:
            handles[k + 1] = start(k + 1, bufs[(k + 1) % 2], sems[(k + 1) % 2])
        handles[k].wait()
        accumulate(bufs[k % 2])

    @pl.when(w == 0)
    def _tail():
        pltpu.sync_copy(buf_hbm.at[pl.ds(_W - _CH, _CH), :], cbuf0)
        accumulate(cbuf0)

    pltpu.sync_copy(acc_v, spart.at[s_ax])
    plsc.subcore_barrier()

    @pl.when(s_ax == 0)
    def _combine():
        pltpu.sync_copy(spart, comb_v)

        def jbody(j, _):
            o = j * _LANES
            v = comb_v[0, pl.ds(o, _LANES)]
            for r in range(1, _NS):
                v = v + comb_v[r, pl.ds(o, _LANES)]
            out_v[pl.ds(o, _LANES)] = v
            return 0

        lax.fori_loop(0, _JGRID, jbody, 0)
        pltpu.sync_copy(out_v, out_hbm.at[c_ax])


_sc_buf_sum = functools.partial(
    pl.kernel,
    out_type=jax.ShapeDtypeStruct((_NC, _D), jnp.float32),
    mesh=plsc.VectorSubcoreMesh(
        core_axis_name="c", subcore_axis_name="s",
        num_cores=_NC, num_subcores=_NS),
    scratch_types=[
        pltpu.VMEM((_CH, _D), jnp.float32),
        pltpu.VMEM((_CH, _D), jnp.float32),
        pltpu.VMEM((_D,), jnp.float32),
        pltpu.VMEM_SHARED((_NS, _D), jnp.float32),
        pltpu.VMEM((_NS, _D), jnp.float32),
        pltpu.VMEM((_D,), jnp.float32),
        pltpu.SemaphoreType.DMA,
        pltpu.SemaphoreType.DMA,
    ],
)(_sc_body)


_TC_XBLK = 1024
_TC_BBLK = 1184
_TC_GRID = 8 + _R0 // _TC_BBLK


def _tc_body(x_ref, buf_ref, out_ref):
    i = pl.program_id(0)

    @pl.when(i == 0)
    def _init():
        out_ref[...] = jnp.zeros_like(out_ref)

    @pl.when(i < 8)
    def _x_part():
        out_ref[...] += jnp.sum(x_ref[...], axis=0, keepdims=True) * (1.0 / _B)

    @pl.when(i >= 8)
    def _buf_part():
        out_ref[...] += jnp.sum(buf_ref[...], axis=0, keepdims=True)


def _tc_main(x, buf):
    return pl.pallas_call(
        _tc_body,
        grid=(_TC_GRID,),
        in_specs=[
            pl.BlockSpec((_TC_XBLK, _D), lambda i: (jnp.minimum(i, 7), 0)),
            pl.BlockSpec((_TC_BBLK, _D), lambda i: (jnp.maximum(i - 8, 0), 0)),
        ],
        out_specs=pl.BlockSpec((1, _D), lambda i: (0, 0)),
        out_shape=jax.ShapeDtypeStruct((1, _D), jnp.float32),
        compiler_params=pltpu.CompilerParams(
            dimension_semantics=("arbitrary",),
        ),
    )(x, buf)


def _combine_body(pos_ref, bufrow_ref, tcp_ref, scp_ref, out_ref):
    del pos_ref
    total = (tcp_ref[...]
             + scp_ref[pl.ds(0, 1), :]
             + scp_ref[pl.ds(1, 1), :]
             - bufrow_ref[0])
    out_ref[...] = total * (1.0 / _W)


def _combine(pos_arr, buf, tc_p, sc_p):
    buf3 = buf.reshape((_W, 1, _D))
    grid_spec = pltpu.PrefetchScalarGridSpec(
        num_scalar_prefetch=1,
        grid=(1,),
        in_specs=[
            pl.BlockSpec((1, 1, _D), lambda i, pos_ref: (pos_ref[0], 0, 0)),
            pl.BlockSpec((1, _D), lambda i, pos_ref: (0, 0)),
            pl.BlockSpec((_NC, _D), lambda i, pos_ref: (0, 0)),
        ],
        out_specs=pl.BlockSpec((1, _D), lambda i, pos_ref: (0, 0)),
    )
    return pl.pallas_call(
        _combine_body,
        grid_spec=grid_spec,
        out_shape=jax.ShapeDtypeStruct((1, _D), jnp.float32),
    )(pos_arr, buf3, tc_p, sc_p)


def kernel(x, buf, pos):
    pos_arr = jnp.asarray(pos, dtype=jnp.int32).reshape((1,))
    sc_p = _sc_buf_sum(buf)
    tc_p = _tc_main(x, buf)
    out = _combine(pos_arr, buf, tc_p, sc_p)
    return out.reshape((_D,))


# speedup vs baseline: 1.0019x; 1.0019x over previous
import functools

import jax
import jax.numpy as jnp
from jax import lax
from jax.experimental import pallas as pl
from jax.experimental.pallas import tpu as pltpu
from jax.experimental.pallas import tpu_sc as plsc

_W = 10000
_D = 2048
_B = 8192

_R0 = 9984
_CH = 16
_LANES = 16
_JGRID = _D // _LANES


def _sc_body(buf_hbm, out_hbm, cbuf, acc_v):
    c_ax = lax.axis_index("c")
    s_ax = lax.axis_index("s")

    @pl.when((c_ax == 0) & (s_ax == 0))
    def _only_tile():
        pltpu.sync_copy(buf_hbm.at[pl.ds(_R0, _CH), :], cbuf)

        def jbody(j, _):
            o = j * _LANES
            v = cbuf[0, pl.ds(o, _LANES)]
            for r in range(1, _CH):
                v = v + cbuf[r, pl.ds(o, _LANES)]
            acc_v[pl.ds(o, _LANES)] = v
            return 0

        lax.fori_loop(0, _JGRID, jbody, 0)
        pltpu.sync_copy(acc_v, out_hbm.at[0])


_sc_buf_sum = functools.partial(
    pl.kernel,
    out_type=jax.ShapeDtypeStruct((1, _D), jnp.float32),
    mesh=plsc.VectorSubcoreMesh(
        core_axis_name="c", subcore_axis_name="s",
        num_cores=1, num_subcores=16),
    scratch_types=[
        pltpu.VMEM((_CH, _D), jnp.float32),
        pltpu.VMEM((_D,), jnp.float32),
    ],
)(_sc_body)


_TC_XBLK = 1024
_TC_BBLK = 1248
_TC_GRID = 8 + _R0 // _TC_BBLK


def _tc_body(x_ref, buf_ref, out_ref):
    i = pl.program_id(0)

    @pl.when(i == 0)
    def _init():
        out_ref[...] = jnp.zeros_like(out_ref)

    @pl.when(i < 8)
    def _x_part():
        out_ref[...] += jnp.sum(x_ref[...], axis=0, keepdims=True) * (1.0 / _B)

    @pl.when(i >= 8)
    def _buf_part():
        out_ref[...] += jnp.sum(buf_ref[...], axis=0, keepdims=True)


def _tc_main(x, buf):
    return pl.pallas_call(
        _tc_body,
        grid=(_TC_GRID,),
        in_specs=[
            pl.BlockSpec((_TC_XBLK, _D), lambda i: (jnp.minimum(i, 7), 0)),
            pl.BlockSpec((_TC_BBLK, _D), lambda i: (jnp.maximum(i - 8, 0), 0)),
        ],
        out_specs=pl.BlockSpec((1, _D), lambda i: (0, 0)),
        out_shape=jax.ShapeDtypeStruct((1, _D), jnp.float32),
        compiler_params=pltpu.CompilerParams(
            dimension_semantics=("arbitrary",),
        ),
    )(x, buf)


def _combine_body(pos_ref, bufrow_ref, tcp_ref, scp_ref, out_ref):
    del pos_ref
    total = (tcp_ref[...] + scp_ref[...] - bufrow_ref[0])
    out_ref[...] = total * (1.0 / _W)


def _combine(pos_arr, buf, tc_p, sc_p):
    buf3 = buf.reshape((_W, 1, _D))
    grid_spec = pltpu.PrefetchScalarGridSpec(
        num_scalar_prefetch=1,
        grid=(1,),
        in_specs=[
            pl.BlockSpec((1, 1, _D), lambda i, pos_ref: (pos_ref[0], 0, 0)),
            pl.BlockSpec((1, _D), lambda i, pos_ref: (0, 0)),
            pl.BlockSpec((1, _D), lambda i, pos_ref: (0, 0)),
        ],
        out_specs=pl.BlockSpec((1, _D), lambda i, pos_ref: (0, 0)),
    )
    return pl.pallas_call(
        _combine_body,
        grid_spec=grid_spec,
        out_shape=jax.ShapeDtypeStruct((1, _D), jnp.float32),
    )(pos_arr, buf3, tc_p, sc_p)


def kernel(x, buf, pos):
    pos_arr = jnp.asarray(pos, dtype=jnp.int32).reshape((1,))
    sc_p = _sc_buf_sum(buf)
    tc_p = _tc_main(x, buf)
    out = _combine(pos_arr, buf, tc_p, sc_p)
    return out.reshape((_D,))


# speedup vs baseline: 4.5908x; 4.5819x over previous
import jax
import jax.numpy as jnp
from jax.experimental import pallas as pl
from jax.experimental.pallas import tpu as pltpu

_W = 10000
_D = 2048
_B = 8192

_BUF_ROWS = 2000
_X_ROWS = 1024
_GRID = 8


def _body(pos_ref, x_ref, buf_ref, out_ref):
    i = pl.program_id(0)

    @pl.when(i == 0)
    def _init():
        out_ref[...] = jnp.zeros_like(out_ref)

    @pl.when(i < 5)
    def _buf_part():
        out_ref[...] += jnp.sum(buf_ref[...], axis=0, keepdims=True)

    out_ref[...] += jnp.sum(x_ref[...], axis=0, keepdims=True) * (1.0 / _B)

    pos = pos_ref[0]

    @pl.when(i == pos // _BUF_ROWS)
    def _subtract_old_row():
        out_ref[...] -= buf_ref[pl.ds(pos % _BUF_ROWS, 1), :]

    @pl.when(i == _GRID - 1)
    def _finish():
        out_ref[...] *= (1.0 / _W)


def kernel(x, buf, pos):
    pos_arr = jnp.asarray(pos, dtype=jnp.int32).reshape((1,))
    out = pl.pallas_call(
        _body,
        grid=(_GRID,),
        in_specs=[
            pl.BlockSpec(memory_space=pltpu.SMEM),
            pl.BlockSpec((_X_ROWS, _D), lambda i: (i, 0)),
            pl.BlockSpec((_BUF_ROWS, _D), lambda i: (jnp.minimum(i, 4), 0)),
        ],
        out_specs=pl.BlockSpec((1, _D), lambda i: (0, 0)),
        out_shape=jax.ShapeDtypeStruct((1, _D), jnp.float32),
        compiler_params=pltpu.CompilerParams(
            dimension_semantics=("arbitrary",),
        ),
    )(pos_arr, x, buf)
    return out.reshape((_D,))
